# baseline (speedup 1.0000x reference)
import functools

import jax
import jax.numpy as jnp
from jax.experimental import pallas as pl
from jax.experimental.pallas import tpu as pltpu
from jax.experimental.pallas import tpu_sc as plsc

TILE = 128
SCW = 128


def _gate_route_body(x_ref, wgt_ref, gates_ref, pos_ref, te_ref):
    B, D = x_ref.shape
    E = wgt_ref.shape[0]
    P = 2 * B
    logitsT = jax.lax.dot_general(
        wgt_ref[...], x_ref[...], (((1,), (1,)), ((), ())),
        preferred_element_type=jnp.float32)
    NEG = jnp.float32(-1e30)
    m1 = jnp.full((1, B), NEG, jnp.float32)
    a1 = jnp.zeros((1, B), jnp.int32)
    for e in range(E):
        v = logitsT[e:e + 1, :]
        take = v > m1
        a1 = jnp.where(take, e, a1)
        m1 = jnp.where(take, v, m1)
    m2 = jnp.full((1, B), NEG, jnp.float32)
    a2 = jnp.zeros((1, B), jnp.int32)
    for e in range(E):
        v = logitsT[e:e + 1, :]
        take = (v > m2) & (a1 != e)
        a2 = jnp.where(take, e, a2)
        m2 = jnp.where(take, v, m2)
    g1 = 1.0 / (1.0 + jnp.exp(m2 - m1))
    gates_ref[...] = jnp.concatenate([g1, 1.0 - g1], axis=0)

    rows = []
    for e in range(E):
        rows.append(jnp.concatenate(
            [(a1 == e), (a2 == e)], axis=1).astype(jnp.float32))
    oh = jnp.concatenate(rows, axis=0)

    r = oh
    k = 1
    while k < P:
        r = r + jnp.concatenate(
            [jnp.zeros((E, k), jnp.float32), r[:, :P - k]], axis=1)
        k *= 2
    c = r[:, P - 1:P]
    pc = jnp.floor((c + (TILE - 1)) / TILE) * TILE
    inc = pc
    for k in (1, 2, 4):
        inc = inc + jnp.concatenate(
            [jnp.zeros((k, 1), jnp.float32), inc[:E - k]], axis=0)
    poff = inc - pc
    posf = jnp.sum(oh * (r - 1.0 + poff), axis=0, keepdims=True)
    pos_ref[...] = posf.astype(jnp.int32)

    pend = poff + pc
    starts = jax.lax.broadcasted_iota(
        jnp.int32, (1, te_ref.shape[1]), 1).astype(jnp.float32) * TILE
    tecnt = jnp.sum((pend <= starts).astype(jnp.int32), axis=0, keepdims=True)
    te_ref[...] = jnp.minimum(tecnt, E - 1)


def _gate_route(x, w_gateT, nt_pad):
    B = x.shape[0]
    E = w_gateT.shape[0]
    return pl.pallas_call(
        _gate_route_body,
        out_shape=(
            jax.ShapeDtypeStruct((2, B), jnp.float32),
            jax.ShapeDtypeStruct((1, 2 * B), jnp.int32),
            jax.ShapeDtypeStruct((1, nt_pad), jnp.int32),
        ),
    )(x, w_gateT)


def _sc_mesh():
    return plsc.VectorSubcoreMesh(core_axis_name="c", subcore_axis_name="s")


def _sc_scatter_rows(xi, pos2, npad):
    B, W = xi.shape

    @functools.partial(
        pl.kernel,
        out_type=jax.ShapeDtypeStruct((npad, W), jnp.int32),
        mesh=_sc_mesh())
    def scatter_kernel(x_hbm, i_hbm, o_hbm):
        def body(x_vmem, i_vmem):
            pltpu.sync_copy(x_vmem, o_hbm.at[i_vmem.at[0]])
            pltpu.sync_copy(x_vmem, o_hbm.at[i_vmem.at[1]])

        pltpu.emit_pipeline(
            body,
            grid=(B // SCW,),
            in_specs=[
                pl.BlockSpec((SCW, W), lambda i: (i, 0)),
                pl.BlockSpec((2, SCW), lambda i: (0, i)),
            ],
            out_specs=[],
            core_axis_name=("c", "s"),
            dimension_semantics=(pltpu.PARALLEL,),
        )(x_hbm, i_hbm)

    return scatter_kernel(xi, pos2)


def _sc_gather_rows(buf, pos):
    W = buf.shape[1]
    P = pos.shape[1]

    @functools.partial(
        pl.kernel,
        out_type=jax.ShapeDtypeStruct((P, W), jnp.int32),
        mesh=_sc_mesh())
    def gather_kernel(x_hbm, i_hbm, o_hbm):
        def body(i_vmem, o_vmem):
            pltpu.sync_copy(x_hbm.at[i_vmem.at[0]], o_vmem)

        pltpu.emit_pipeline(
            body,
            grid=(P // SCW,),
            in_specs=[pl.BlockSpec((1, SCW), lambda i: (0, i))],
            out_specs=[pl.BlockSpec((SCW, W), lambda i: (i, 0))],
            core_axis_name=("c", "s"),
            dimension_semantics=(pltpu.PARALLEL,),
        )(i_hbm, o_hbm)

    return gather_kernel(buf, pos)


def _expert_body(te_ref, buf_ref, w1_ref, b1_ref, w2_ref, b2_ref, o_ref):
    xt = buf_ref[...]
    h = jnp.dot(xt, w1_ref[0], preferred_element_type=jnp.float32)
    h = jnp.maximum(h + b1_ref[0], 0.0).astype(jnp.bfloat16)
    o = jnp.dot(h, w2_ref[0], preferred_element_type=jnp.float32)
    o = o + b2_ref[0]
    m = jnp.max(o, axis=1, keepdims=True)
    p = jnp.exp(o - m)
    o_ref[...] = (p / jnp.sum(p, axis=1, keepdims=True)).astype(jnp.bfloat16)


def _expert_mlp(te, buf, W1bf, b1, W2bf, b2, nt):
    npad, D = buf.shape
    E, _, H = W1bf.shape
    MOUT = W2bf.shape[2]
    grid_spec = pltpu.PrefetchScalarGridSpec(
        num_scalar_prefetch=1,
        grid=(nt,),
        in_specs=[
            pl.BlockSpec((TILE, D), lambda i, te_r: (i, 0)),
            pl.BlockSpec((1, D, H), lambda i, te_r: (te_r[i], 0, 0)),
            pl.BlockSpec((1, 1, H), lambda i, te_r: (te_r[i], 0, 0)),
            pl.BlockSpec((1, H, MOUT), lambda i, te_r: (te_r[i], 0, 0)),
            pl.BlockSpec((1, 1, MOUT), lambda i, te_r: (te_r[i], 0, 0)),
        ],
        out_specs=pl.BlockSpec((TILE, MOUT), lambda i, te_r: (i, 0)),
    )
    return pl.pallas_call(
        _expert_body,
        grid_spec=grid_spec,
        out_shape=jax.ShapeDtypeStruct((npad, MOUT), jnp.bfloat16),
    )(te, buf, W1bf, b1, W2bf, b2)


def _combine_heads_body(op_ref, g_ref, w11_ref, b11_ref, w12_ref,
                        b12_ref, w21_ref, b21_ref, w22_ref, b22_ref,
                        y1_ref, y2_ref):
    g = g_ref[...]
    moe = (op_ref[0].astype(jnp.float32) * g[:, 0:1]
           + op_ref[1].astype(jnp.float32) * g[:, 1:2])
    mo = moe.astype(jnp.bfloat16)
    h1 = jnp.dot(mo, w11_ref[...], preferred_element_type=jnp.float32)
    h1 = jnp.maximum(h1 + b11_ref[...], 0.0).astype(jnp.bfloat16)
    y1_ref[...] = (jnp.dot(h1, w12_ref[...],
                           preferred_element_type=jnp.float32) + b12_ref[...])
    h2 = jnp.dot(mo, w21_ref[...], preferred_element_type=jnp.float32)
    h2 = jnp.maximum(h2 + b21_ref[...], 0.0).astype(jnp.bfloat16)
    y2_ref[...] = (jnp.dot(h2, w22_ref[...],
                           preferred_element_type=jnp.float32) + b22_ref[...])


def _combine_heads(op, gT, hw11, hb11, hw12, hb12, hw21, hb21,
                   hw22, hb22):
    _, B, MOUT = op.shape
    MH = hw11.shape[1]
    OUT = hw12.shape[1]
    TT = 256
    grid = (B // TT,)
    y1, y2 = pl.pallas_call(
        _combine_heads_body,
        grid=grid,
        in_specs=[
            pl.BlockSpec((2, TT, MOUT), lambda i: (0, i, 0)),
            pl.BlockSpec((TT, 2), lambda i: (i, 0)),
            pl.BlockSpec((MOUT, MH), lambda i: (0, 0)),
            pl.BlockSpec((1, MH), lambda i: (0, 0)),
            pl.BlockSpec((MH, OUT), lambda i: (0, 0)),
            pl.BlockSpec((1, OUT), lambda i: (0, 0)),
            pl.BlockSpec((MOUT, MH), lambda i: (0, 0)),
            pl.BlockSpec((1, MH), lambda i: (0, 0)),
            pl.BlockSpec((MH, OUT), lambda i: (0, 0)),
            pl.BlockSpec((1, OUT), lambda i: (0, 0)),
        ],
        out_specs=[
            pl.BlockSpec((TT, OUT), lambda i: (i, 0)),
            pl.BlockSpec((TT, OUT), lambda i: (i, 0)),
        ],
        out_shape=(
            jax.ShapeDtypeStruct((B, OUT), jnp.float32),
            jax.ShapeDtypeStruct((B, OUT), jnp.float32),
        ),
    )(op, gT, hw11, hb11, hw12, hb12, hw21, hb21, hw22, hb22)
    return y1, y2


def kernel(x, w_gate, W1, b1, W2, b2, m1_W1, m1_b1, m1_W2, m1_b2,
           m2_W1, m2_b1, m2_W2, m2_b2):
    B, D = x.shape
    E = w_gate.shape[1]
    H = W1.shape[2]
    MOUT = W2.shape[2]
    MH = m1_W1.shape[1]
    OUT = m1_W2.shape[1]
    npad = ((2 * B + E * (TILE - 1)) + TILE - 1) // TILE * TILE
    nt = npad // TILE
    nt_pad = ((nt + 63) // 64) * 64

    gates, pos, te = _gate_route(x, w_gate.T, nt_pad)
    te1 = te.reshape(nt_pad)[:nt]

    xi = jax.lax.bitcast_convert_type(
        x.astype(jnp.bfloat16).reshape(B, D // 2, 2), jnp.int32)
    bufi = _sc_scatter_rows(xi, pos.reshape(2, B), npad)
    buf = jax.lax.bitcast_convert_type(
        bufi, jnp.bfloat16).reshape(npad, D)
    obuf = _expert_mlp(
        te1, buf, W1.astype(jnp.bfloat16), b1.reshape(E, 1, H),
        W2.astype(jnp.bfloat16), b2.reshape(E, 1, MOUT), nt)
    obi = jax.lax.bitcast_convert_type(
        obuf.reshape(npad, MOUT // 2, 2), jnp.int32)
    opi = _sc_gather_rows(obi, pos)
    op = jax.lax.bitcast_convert_type(
        opi, jnp.bfloat16).reshape(2, B, MOUT)

    y1, y2 = _combine_heads(
        op, gates.T,
        m1_W1.astype(jnp.bfloat16), m1_b1.reshape(1, MH),
        m1_W2.astype(jnp.bfloat16), m1_b2.reshape(1, OUT),
        m2_W1.astype(jnp.bfloat16), m2_b1.reshape(1, MH),
        m2_W2.astype(jnp.bfloat16), m2_b2.reshape(1, OUT))
    return (y1, y2)


# speedup vs baseline: 2.8931x; 2.8931x over previous
import functools

import jax
import jax.numpy as jnp
from jax.experimental import pallas as pl
from jax.experimental.pallas import tpu as pltpu
from jax.experimental.pallas import tpu_sc as plsc

TILE = 128
SCW = 128


def _gate_route_body(x_ref, wgt_ref, gates_ref, pos_ref, te_ref):
    B, D = x_ref.shape
    E = wgt_ref.shape[0]
    P = 2 * B
    logitsT = jax.lax.dot_general(
        wgt_ref[...], x_ref[...], (((1,), (1,)), ((), ())),
        preferred_element_type=jnp.float32)
    NEG = jnp.float32(-1e30)
    m1 = jnp.full((1, B), NEG, jnp.float32)
    a1 = jnp.zeros((1, B), jnp.int32)
    for e in range(E):
        v = logitsT[e:e + 1, :]
        take = v > m1
        a1 = jnp.where(take, e, a1)
        m1 = jnp.where(take, v, m1)
    m2 = jnp.full((1, B), NEG, jnp.float32)
    a2 = jnp.zeros((1, B), jnp.int32)
    for e in range(E):
        v = logitsT[e:e + 1, :]
        take = (v > m2) & (a1 != e)
        a2 = jnp.where(take, e, a2)
        m2 = jnp.where(take, v, m2)
    g1 = 1.0 / (1.0 + jnp.exp(m2 - m1))
    gates_ref[...] = jnp.concatenate([g1, 1.0 - g1], axis=0)

    rows = []
    for e in range(E):
        rows.append(jnp.concatenate(
            [(a1 == e), (a2 == e)], axis=1).astype(jnp.float32))
    oh = jnp.concatenate(rows, axis=0)

    r = oh
    k = 1
    while k < P:
        r = r + jnp.concatenate(
            [jnp.zeros((E, k), jnp.float32), r[:, :P - k]], axis=1)
        k *= 2
    c = r[:, P - 1:P]
    pc = jnp.floor((c + (TILE - 1)) / TILE) * TILE
    inc = pc
    for k in (1, 2, 4):
        inc = inc + jnp.concatenate(
            [jnp.zeros((k, 1), jnp.float32), inc[:E - k]], axis=0)
    poff = inc - pc
    posf = jnp.sum(oh * (r - 1.0 + poff), axis=0, keepdims=True)
    pos_ref[...] = posf.astype(jnp.int32)

    pend = poff + pc
    starts = jax.lax.broadcasted_iota(
        jnp.int32, (1, te_ref.shape[1]), 1).astype(jnp.float32) * TILE
    tecnt = jnp.sum((pend <= starts).astype(jnp.int32), axis=0, keepdims=True)
    te_ref[...] = jnp.minimum(tecnt, E - 1)


def _gate_route(x, w_gateT, nt_pad):
    B = x.shape[0]
    E = w_gateT.shape[0]
    return pl.pallas_call(
        _gate_route_body,
        out_shape=(
            jax.ShapeDtypeStruct((2, B), jnp.float32),
            jax.ShapeDtypeStruct((1, 2 * B), jnp.int32),
            jax.ShapeDtypeStruct((1, nt_pad), jnp.int32),
        ),
    )(x, w_gateT)


def _sc_mesh():
    return plsc.VectorSubcoreMesh(core_axis_name="c", subcore_axis_name="s")


def _sc_scatter_rows(xf, pos2, npad):
    B, W = xf.shape

    @functools.partial(
        pl.kernel,
        out_type=jax.ShapeDtypeStruct((npad, W), jnp.float32),
        mesh=_sc_mesh())
    def scatter_kernel(x_hbm, i_hbm, o_hbm):
        def body(x_vmem, i_vmem):
            pltpu.sync_copy(x_vmem, o_hbm.at[i_vmem.at[0]])
            pltpu.sync_copy(x_vmem, o_hbm.at[i_vmem.at[1]])

        pltpu.emit_pipeline(
            body,
            grid=(B // SCW,),
            in_specs=[
                pl.BlockSpec((SCW, W), lambda i: (i, 0)),
                pl.BlockSpec((2, SCW), lambda i: (0, i)),
            ],
            out_specs=[],
            core_axis_name=("c", "s"),
            dimension_semantics=(pltpu.PARALLEL,),
        )(x_hbm, i_hbm)

    return scatter_kernel(xf, pos2)


def _sc_gather_rows(buf, pos):
    W = buf.shape[1]
    P = pos.shape[1]

    @functools.partial(
        pl.kernel,
        out_type=jax.ShapeDtypeStruct((P, W), jnp.float32),
        mesh=_sc_mesh())
    def gather_kernel(x_hbm, i_hbm, o_hbm):
        def body(i_vmem, o_vmem):
            pltpu.sync_copy(x_hbm.at[i_vmem.at[0]], o_vmem)

        pltpu.emit_pipeline(
            body,
            grid=(P // SCW,),
            in_specs=[pl.BlockSpec((1, SCW), lambda i: (0, i))],
            out_specs=[pl.BlockSpec((SCW, W), lambda i: (i, 0))],
            core_axis_name=("c", "s"),
            dimension_semantics=(pltpu.PARALLEL,),
        )(i_hbm, o_hbm)

    return gather_kernel(buf, pos)


def _expert_body(te_ref, bufl_ref, bufr_ref, w1_ref, b1_ref, w2_ref, b2_ref,
                 ol_ref, or_ref):
    xt = jnp.concatenate([bufl_ref[...], bufr_ref[...]], axis=1)
    xt = xt.astype(w1_ref.dtype)
    h = jnp.dot(xt, w1_ref[0], preferred_element_type=jnp.float32)
    h = jnp.maximum(h + b1_ref[0], 0.0).astype(w1_ref.dtype)
    o = jnp.dot(h, w2_ref[0], preferred_element_type=jnp.float32)
    o = o + b2_ref[0]
    m = jnp.max(o, axis=1, keepdims=True)
    p = jnp.exp(o - m)
    o = p / jnp.sum(p, axis=1, keepdims=True)
    half = o.shape[1] // 2
    ol_ref[...] = o[:, :half]
    or_ref[...] = o[:, half:]


def _expert_mlp(te, bufl, bufr, W1, b1, W2, b2, nt):
    npad, Dh = bufl.shape
    E, D, H = W1.shape
    MOUT = W2.shape[2]
    grid_spec = pltpu.PrefetchScalarGridSpec(
        num_scalar_prefetch=1,
        grid=(nt,),
        in_specs=[
            pl.BlockSpec((TILE, Dh), lambda i, te_r: (i, 0)),
            pl.BlockSpec((TILE, Dh), lambda i, te_r: (i, 0)),
            pl.BlockSpec((1, D, H), lambda i, te_r: (te_r[i], 0, 0)),
            pl.BlockSpec((1, 1, H), lambda i, te_r: (te_r[i], 0, 0)),
            pl.BlockSpec((1, H, MOUT), lambda i, te_r: (te_r[i], 0, 0)),
            pl.BlockSpec((1, 1, MOUT), lambda i, te_r: (te_r[i], 0, 0)),
        ],
        out_specs=[
            pl.BlockSpec((TILE, MOUT // 2), lambda i, te_r: (i, 0)),
            pl.BlockSpec((TILE, MOUT // 2), lambda i, te_r: (i, 0)),
        ],
    )
    return pl.pallas_call(
        _expert_body,
        grid_spec=grid_spec,
        out_shape=(
            jax.ShapeDtypeStruct((npad, MOUT // 2), jnp.float32),
            jax.ShapeDtypeStruct((npad, MOUT // 2), jnp.float32),
        ),
    )(te, bufl, bufr, W1, b1, W2, b2)


def _combine_heads_body(opl_ref, opr_ref, g_ref, w11_ref, b11_ref, w12_ref,
                        b12_ref, w21_ref, b21_ref, w22_ref, b22_ref,
                        y1_ref, y2_ref):
    g = g_ref[...]
    op0 = jnp.concatenate([opl_ref[0], opr_ref[0]], axis=1)
    op1 = jnp.concatenate([opl_ref[1], opr_ref[1]], axis=1)
    moe = op0 * g[:, 0:1] + op1 * g[:, 1:2]
    mo = moe.astype(w11_ref.dtype)
    h1 = jnp.dot(mo, w11_ref[...], preferred_element_type=jnp.float32)
    h1 = jnp.maximum(h1 + b11_ref[...], 0.0).astype(w11_ref.dtype)
    y1_ref[...] = (jnp.dot(h1, w12_ref[...],
                           preferred_element_type=jnp.float32) + b12_ref[...])
    h2 = jnp.dot(mo, w21_ref[...], preferred_element_type=jnp.float32)
    h2 = jnp.maximum(h2 + b21_ref[...], 0.0).astype(w11_ref.dtype)
    y2_ref[...] = (jnp.dot(h2, w22_ref[...],
                           preferred_element_type=jnp.float32) + b22_ref[...])


def _combine_heads(opl, opr, gT, hw11, hb11, hw12, hb12, hw21, hb21,
                   hw22, hb22):
    _, B, MOUTH = opl.shape
    MOUT = 2 * MOUTH
    MH = hw11.shape[1]
    OUT = hw12.shape[1]
    TT = 256
    grid = (B // TT,)
    y1, y2 = pl.pallas_call(
        _combine_heads_body,
        grid=grid,
        in_specs=[
            pl.BlockSpec((2, TT, MOUTH), lambda i: (0, i, 0)),
            pl.BlockSpec((2, TT, MOUTH), lambda i: (0, i, 0)),
            pl.BlockSpec((TT, 2), lambda i: (i, 0)),
            pl.BlockSpec((MOUT, MH), lambda i: (0, 0)),
            pl.BlockSpec((1, MH), lambda i: (0, 0)),
            pl.BlockSpec((MH, OUT), lambda i: (0, 0)),
            pl.BlockSpec((1, OUT), lambda i: (0, 0)),
            pl.BlockSpec((MOUT, MH), lambda i: (0, 0)),
            pl.BlockSpec((1, MH), lambda i: (0, 0)),
            pl.BlockSpec((MH, OUT), lambda i: (0, 0)),
            pl.BlockSpec((1, OUT), lambda i: (0, 0)),
        ],
        out_specs=[
            pl.BlockSpec((TT, OUT), lambda i: (i, 0)),
            pl.BlockSpec((TT, OUT), lambda i: (i, 0)),
        ],
        out_shape=(
            jax.ShapeDtypeStruct((B, OUT), jnp.float32),
            jax.ShapeDtypeStruct((B, OUT), jnp.float32),
        ),
    )(opl, opr, gT, hw11, hb11, hw12, hb12, hw21, hb21, hw22, hb22)
    return y1, y2


def kernel(x, w_gate, W1, b1, W2, b2, m1_W1, m1_b1, m1_W2, m1_b2,
           m2_W1, m2_b1, m2_W2, m2_b2):
    B, D = x.shape
    E = w_gate.shape[1]
    H = W1.shape[2]
    MOUT = W2.shape[2]
    MH = m1_W1.shape[1]
    OUT = m1_W2.shape[1]
    npad = ((2 * B + E * (TILE - 1)) + TILE - 1) // TILE * TILE
    nt = npad // TILE
    nt_pad = ((nt + 63) // 64) * 64

    gates, pos, te = _gate_route(x, w_gate.T, nt_pad)
    te1 = te.reshape(nt_pad)[:nt]

    Dh = D // 2
    pos2 = pos.reshape(2, B)
    bufl = _sc_scatter_rows(x[:, :Dh], pos2, npad)
    bufr = _sc_scatter_rows(x[:, Dh:], pos2, npad)
    obufl, obufr = _expert_mlp(
        te1, bufl, bufr, W1, b1.reshape(E, 1, H), W2, b2.reshape(E, 1, MOUT),
        nt)
    opl = _sc_gather_rows(obufl, pos)
    opr = _sc_gather_rows(obufr, pos)

    y1, y2 = _combine_heads(
        opl.reshape(2, B, MOUT // 2), opr.reshape(2, B, MOUT // 2), gates.T,
        m1_W1, m1_b1.reshape(1, MH), m1_W2, m1_b2.reshape(1, OUT),
        m2_W1, m2_b1.reshape(1, MH), m2_W2, m2_b2.reshape(1, OUT))
    return (y1, y2)


# speedup vs baseline: 3.2204x; 1.1132x over previous
import functools

import jax
import jax.numpy as jnp
from jax.experimental import pallas as pl
from jax.experimental.pallas import tpu as pltpu
from jax.experimental.pallas import tpu_sc as plsc

TILE = 128
SCW = 128


def _gate_route_body(x_ref, wgt_ref, gates_ref, pos_ref, te_ref):
    B, D = x_ref.shape
    E = wgt_ref.shape[0]
    P = 2 * B
    logitsT = jax.lax.dot_general(
        wgt_ref[...], x_ref[...], (((1,), (1,)), ((), ())),
        preferred_element_type=jnp.float32)
    NEG = jnp.float32(-1e30)
    m1 = jnp.full((1, B), NEG, jnp.float32)
    a1 = jnp.zeros((1, B), jnp.int32)
    for e in range(E):
        v = logitsT[e:e + 1, :]
        take = v > m1
        a1 = jnp.where(take, e, a1)
        m1 = jnp.where(take, v, m1)
    m2 = jnp.full((1, B), NEG, jnp.float32)
    a2 = jnp.zeros((1, B), jnp.int32)
    for e in range(E):
        v = logitsT[e:e + 1, :]
        take = (v > m2) & (a1 != e)
        a2 = jnp.where(take, e, a2)
        m2 = jnp.where(take, v, m2)
    g1 = 1.0 / (1.0 + jnp.exp(m2 - m1))
    gates_ref[...] = jnp.transpose(
        jnp.concatenate([g1, 1.0 - g1], axis=0))

    rows = []
    for e in range(E):
        rows.append(jnp.concatenate(
            [(a1 == e), (a2 == e)], axis=1).astype(jnp.float32))
    oh = jnp.concatenate(rows, axis=0)

    r = oh
    k = 1
    while k < P:
        r = r + jnp.concatenate(
            [jnp.zeros((E, k), jnp.float32), r[:, :P - k]], axis=1)
        k *= 2
    c = r[:, P - 1:P]
    pc = jnp.floor((c + (TILE - 1)) / TILE) * TILE
    inc = pc
    for k in (1, 2, 4):
        inc = inc + jnp.concatenate(
            [jnp.zeros((k, 1), jnp.float32), inc[:E - k]], axis=0)
    poff = inc - pc
    posf = jnp.sum(oh * (r - 1.0 + poff), axis=0, keepdims=True)
    pos_ref[...] = jnp.concatenate(
        [posf[:, :B], posf[:, B:]], axis=0).astype(jnp.int32)

    pend = poff + pc
    starts = jax.lax.broadcasted_iota(
        jnp.int32, (1, te_ref.shape[1]), 1).astype(jnp.float32) * TILE
    tecnt = jnp.sum((pend <= starts).astype(jnp.int32), axis=0, keepdims=True)
    te_ref[...] = jnp.minimum(tecnt, E - 1)


def _gate_route(x, w_gateT, nt_pad):
    B = x.shape[0]
    E = w_gateT.shape[0]
    return pl.pallas_call(
        _gate_route_body,
        out_shape=(
            jax.ShapeDtypeStruct((B, 2), jnp.float32),
            jax.ShapeDtypeStruct((2, B), jnp.int32),
            jax.ShapeDtypeStruct((1, nt_pad), jnp.int32),
        ),
    )(x, w_gateT)


def _sc_mesh():
    return plsc.VectorSubcoreMesh(core_axis_name="c", subcore_axis_name="s")


def _sc_scatter_rows(xl, xr, pos2, npad):
    B, W = xl.shape

    @functools.partial(
        pl.kernel,
        out_type=(jax.ShapeDtypeStruct((npad, W), jnp.float32),
                  jax.ShapeDtypeStruct((npad, W), jnp.float32)),
        mesh=_sc_mesh())
    def scatter_kernel(xl_hbm, xr_hbm, i_hbm, ol_hbm, or_hbm):
        cid = jax.lax.axis_index("c")

        def scat(x_hbm, o_hbm):
            def body(x_vmem, i_vmem):
                pltpu.sync_copy(x_vmem, o_hbm.at[i_vmem.at[0]])
                pltpu.sync_copy(x_vmem, o_hbm.at[i_vmem.at[1]])

            pltpu.emit_pipeline(
                body,
                grid=(B // SCW,),
                in_specs=[
                    pl.BlockSpec((SCW, W), lambda i: (i, 0)),
                    pl.BlockSpec((2, SCW), lambda i: (0, i)),
                ],
                out_specs=[],
                core_axis_name="s",
                dimension_semantics=(pltpu.PARALLEL,),
            )(x_hbm, i_hbm)

        @pl.when(cid == 0)
        def _():
            scat(xl_hbm, ol_hbm)

        @pl.when(cid == 1)
        def _():
            scat(xr_hbm, or_hbm)

    return scatter_kernel(xl, xr, pos2)


def _sc_gather_rows(bufl, bufr, pos2):
    W = bufl.shape[1]
    K, B = pos2.shape
    P = K * B

    @functools.partial(
        pl.kernel,
        out_type=(jax.ShapeDtypeStruct((P, W), jnp.float32),
                  jax.ShapeDtypeStruct((P, W), jnp.float32)),
        mesh=_sc_mesh())
    def gather_kernel(xl_hbm, xr_hbm, i_hbm, ol_hbm, or_hbm):
        cid = jax.lax.axis_index("c")
        nblk = B // SCW

        def gath(x_hbm, o_hbm):
            def body(i_vmem, o_vmem):
                pltpu.sync_copy(x_hbm.at[i_vmem.at[0]], o_vmem)

            pltpu.emit_pipeline(
                body,
                grid=(P // SCW,),
                in_specs=[pl.BlockSpec(
                    (1, SCW),
                    lambda i: (jax.lax.div(i, nblk), jax.lax.rem(i, nblk)))],
                out_specs=[pl.BlockSpec((SCW, W), lambda i: (i, 0))],
                core_axis_name="s",
                dimension_semantics=(pltpu.PARALLEL,),
            )(i_hbm, o_hbm)

        @pl.when(cid == 0)
        def _():
            gath(xl_hbm, ol_hbm)

        @pl.when(cid == 1)
        def _():
            gath(xr_hbm, or_hbm)

    return gather_kernel(bufl, bufr, pos2)


def _expert_body(te_ref, bufl_ref, bufr_ref, w1_ref, b1_ref, w2_ref, b2_ref,
                 ol_ref, or_ref):
    xt = jnp.concatenate([bufl_ref[...], bufr_ref[...]], axis=1)
    xt = xt.astype(w1_ref.dtype)
    h = jnp.dot(xt, w1_ref[0], preferred_element_type=jnp.float32)
    h = jnp.maximum(h + b1_ref[0], 0.0).astype(w1_ref.dtype)
    o = jnp.dot(h, w2_ref[0], preferred_element_type=jnp.float32)
    o = o + b2_ref[0]
    m = jnp.max(o, axis=1, keepdims=True)
    p = jnp.exp(o - m)
    o = p / jnp.sum(p, axis=1, keepdims=True)
    half = o.shape[1] // 2
    ol_ref[...] = o[:, :half]
    or_ref[...] = o[:, half:]


def _expert_mlp(te, bufl, bufr, W1, b1, W2, b2, nt):
    npad, Dh = bufl.shape
    E, D, H = W1.shape
    MOUT = W2.shape[2]
    grid_spec = pltpu.PrefetchScalarGridSpec(
        num_scalar_prefetch=1,
        grid=(nt,),
        in_specs=[
            pl.BlockSpec((TILE, Dh), lambda i, te_r: (i, 0)),
            pl.BlockSpec((TILE, Dh), lambda i, te_r: (i, 0)),
            pl.BlockSpec((1, D, H), lambda i, te_r: (te_r[i], 0, 0)),
            pl.BlockSpec((1, 1, H), lambda i, te_r: (te_r[i], 0, 0)),
            pl.BlockSpec((1, H, MOUT), lambda i, te_r: (te_r[i], 0, 0)),
            pl.BlockSpec((1, 1, MOUT), lambda i, te_r: (te_r[i], 0, 0)),
        ],
        out_specs=[
            pl.BlockSpec((TILE, MOUT // 2), lambda i, te_r: (i, 0)),
            pl.BlockSpec((TILE, MOUT // 2), lambda i, te_r: (i, 0)),
        ],
    )
    return pl.pallas_call(
        _expert_body,
        grid_spec=grid_spec,
        out_shape=(
            jax.ShapeDtypeStruct((npad, MOUT // 2), jnp.float32),
            jax.ShapeDtypeStruct((npad, MOUT // 2), jnp.float32),
        ),
    )(te, bufl, bufr, W1, b1, W2, b2)


def _combine_heads_body(opl_ref, opr_ref, g_ref, w11_ref, b11_ref, w12_ref,
                        b12_ref, w21_ref, b21_ref, w22_ref, b22_ref,
                        y1_ref, y2_ref):
    g = g_ref[...]
    op0 = jnp.concatenate([opl_ref[0], opr_ref[0]], axis=1)
    op1 = jnp.concatenate([opl_ref[1], opr_ref[1]], axis=1)
    moe = op0 * g[:, 0:1] + op1 * g[:, 1:2]
    mo = moe.astype(w11_ref.dtype)
    h1 = jnp.dot(mo, w11_ref[...], preferred_element_type=jnp.float32)
    h1 = jnp.maximum(h1 + b11_ref[...], 0.0).astype(w11_ref.dtype)
    y1_ref[...] = (jnp.dot(h1, w12_ref[...],
                           preferred_element_type=jnp.float32) + b12_ref[...])
    h2 = jnp.dot(mo, w21_ref[...], preferred_element_type=jnp.float32)
    h2 = jnp.maximum(h2 + b21_ref[...], 0.0).astype(w11_ref.dtype)
    y2_ref[...] = (jnp.dot(h2, w22_ref[...],
                           preferred_element_type=jnp.float32) + b22_ref[...])


def _combine_heads(opl, opr, gT, hw11, hb11, hw12, hb12, hw21, hb21,
                   hw22, hb22):
    _, B, MOUTH = opl.shape
    MOUT = 2 * MOUTH
    MH = hw11.shape[1]
    OUT = hw12.shape[1]
    TT = 256
    grid = (B // TT,)
    y1, y2 = pl.pallas_call(
        _combine_heads_body,
        grid=grid,
        in_specs=[
            pl.BlockSpec((2, TT, MOUTH), lambda i: (0, i, 0)),
            pl.BlockSpec((2, TT, MOUTH), lambda i: (0, i, 0)),
            pl.BlockSpec((TT, 2), lambda i: (i, 0)),
            pl.BlockSpec((MOUT, MH), lambda i: (0, 0)),
            pl.BlockSpec((1, MH), lambda i: (0, 0)),
            pl.BlockSpec((MH, OUT), lambda i: (0, 0)),
            pl.BlockSpec((1, OUT), lambda i: (0, 0)),
            pl.BlockSpec((MOUT, MH), lambda i: (0, 0)),
            pl.BlockSpec((1, MH), lambda i: (0, 0)),
            pl.BlockSpec((MH, OUT), lambda i: (0, 0)),
            pl.BlockSpec((1, OUT), lambda i: (0, 0)),
        ],
        out_specs=[
            pl.BlockSpec((TT, OUT), lambda i: (i, 0)),
            pl.BlockSpec((TT, OUT), lambda i: (i, 0)),
        ],
        out_shape=(
            jax.ShapeDtypeStruct((B, OUT), jnp.float32),
            jax.ShapeDtypeStruct((B, OUT), jnp.float32),
        ),
    )(opl, opr, gT, hw11, hb11, hw12, hb12, hw21, hb21, hw22, hb22)
    return y1, y2


def kernel(x, w_gate, W1, b1, W2, b2, m1_W1, m1_b1, m1_W2, m1_b2,
           m2_W1, m2_b1, m2_W2, m2_b2):
    B, D = x.shape
    E = w_gate.shape[1]
    H = W1.shape[2]
    MOUT = W2.shape[2]
    MH = m1_W1.shape[1]
    OUT = m1_W2.shape[1]
    npad = ((2 * B + E * (TILE - 1)) + TILE - 1) // TILE * TILE
    nt = npad // TILE
    nt_pad = ((nt + 63) // 64) * 64

    gates, pos, te = _gate_route(x, w_gate.T, nt_pad)
    te1 = te.reshape(nt_pad)[:nt]

    Dh = D // 2
    bufl, bufr = _sc_scatter_rows(x[:, :Dh], x[:, Dh:], pos, npad)
    obufl, obufr = _expert_mlp(
        te1, bufl, bufr, W1, b1.reshape(E, 1, H), W2, b2.reshape(E, 1, MOUT),
        nt)
    opl, opr = _sc_gather_rows(obufl, obufr, pos)

    y1, y2 = _combine_heads(
        opl.reshape(2, B, MOUT // 2), opr.reshape(2, B, MOUT // 2), gates,
        m1_W1, m1_b1.reshape(1, MH), m1_W2, m1_b2.reshape(1, OUT),
        m2_W1, m2_b1.reshape(1, MH), m2_W2, m2_b2.reshape(1, OUT))
    return (y1, y2)


# speedup vs baseline: 3.4072x; 1.0580x over previous
import functools

import jax
import jax.numpy as jnp
from jax.experimental import pallas as pl
from jax.experimental.pallas import tpu as pltpu
from jax.experimental.pallas import tpu_sc as plsc

TILE = 128
SCW = 128


def _gate_route_body(x_ref, wgt_ref, gates_ref, pos_ref, te_ref):
    B, D = x_ref.shape
    E = wgt_ref.shape[0]
    P = 2 * B
    logitsT = jax.lax.dot_general(
        wgt_ref[...], x_ref[...], (((1,), (1,)), ((), ())),
        preferred_element_type=jnp.float32)
    NEG = jnp.float32(-1e30)
    m1 = jnp.full((1, B), NEG, jnp.float32)
    a1 = jnp.zeros((1, B), jnp.int32)
    for e in range(E):
        v = logitsT[e:e + 1, :]
        take = v > m1
        a1 = jnp.where(take, e, a1)
        m1 = jnp.where(take, v, m1)
    m2 = jnp.full((1, B), NEG, jnp.float32)
    a2 = jnp.zeros((1, B), jnp.int32)
    for e in range(E):
        v = logitsT[e:e + 1, :]
        take = (v > m2) & (a1 != e)
        a2 = jnp.where(take, e, a2)
        m2 = jnp.where(take, v, m2)
    g1 = 1.0 / (1.0 + jnp.exp(m2 - m1))
    gates_ref[...] = jnp.transpose(
        jnp.concatenate([g1, 1.0 - g1], axis=0))

    rows = []
    for e in range(E):
        rows.append(jnp.concatenate(
            [(a1 == e), (a2 == e)], axis=1).astype(jnp.float32))
    oh = jnp.concatenate(rows, axis=0)

    r = oh
    k = 1
    while k < P:
        r = r + jnp.concatenate(
            [jnp.zeros((E, k), jnp.float32), r[:, :P - k]], axis=1)
        k *= 2
    c = r[:, P - 1:P]
    pc = jnp.floor((c + (TILE - 1)) / TILE) * TILE
    inc = pc
    for k in (1, 2, 4):
        inc = inc + jnp.concatenate(
            [jnp.zeros((k, 1), jnp.float32), inc[:E - k]], axis=0)
    poff = inc - pc
    posf = jnp.sum(oh * (r - 1.0 + poff), axis=0, keepdims=True)
    pos_ref[...] = jnp.concatenate(
        [posf[:, :B], posf[:, B:]], axis=0).astype(jnp.int32)

    pend = poff + pc
    starts = jax.lax.broadcasted_iota(
        jnp.int32, (1, te_ref.shape[1]), 1).astype(jnp.float32) * TILE
    tecnt = jnp.sum((pend <= starts).astype(jnp.int32), axis=0, keepdims=True)
    te_ref[...] = jnp.minimum(tecnt, E - 1)


def _gate_route(x, w_gateT, nt_pad):
    B = x.shape[0]
    E = w_gateT.shape[0]
    return pl.pallas_call(
        _gate_route_body,
        out_shape=(
            jax.ShapeDtypeStruct((B, 2), jnp.float32),
            jax.ShapeDtypeStruct((2, B), jnp.int32),
            jax.ShapeDtypeStruct((1, nt_pad), jnp.int32),
        ),
    )(x, w_gateT)


def _sc_mesh():
    return plsc.VectorSubcoreMesh(core_axis_name="c", subcore_axis_name="s")


def _sc_scatter_rows(x, pos2, npad):
    B, D = x.shape
    W = D // 2

    @functools.partial(
        pl.kernel,
        out_type=(jax.ShapeDtypeStruct((npad, W), jnp.float32),
                  jax.ShapeDtypeStruct((npad, W), jnp.float32)),
        mesh=_sc_mesh())
    def scatter_kernel(x_hbm, i_hbm, ol_hbm, or_hbm):
        cid = jax.lax.axis_index("c")

        def scat(col, o_hbm):
            def body(x_vmem, i_vmem):
                pltpu.sync_copy(x_vmem, o_hbm.at[i_vmem.at[0]])
                pltpu.sync_copy(x_vmem, o_hbm.at[i_vmem.at[1]])

            pltpu.emit_pipeline(
                body,
                grid=(B // SCW,),
                in_specs=[
                    pl.BlockSpec((SCW, W), lambda i: (i, col)),
                    pl.BlockSpec((2, SCW), lambda i: (0, i)),
                ],
                out_specs=[],
                core_axis_name="s",
                dimension_semantics=(pltpu.PARALLEL,),
            )(x_hbm, i_hbm)

        @pl.when(cid == 0)
        def _():
            scat(0, ol_hbm)

        @pl.when(cid == 1)
        def _():
            scat(1, or_hbm)

    return scatter_kernel(x, pos2)


def _sc_gather_rows(bufl, bufr, pos2):
    W = bufl.shape[1]
    K, B = pos2.shape
    P = K * B

    @functools.partial(
        pl.kernel,
        out_type=(jax.ShapeDtypeStruct((P, W), jnp.float32),
                  jax.ShapeDtypeStruct((P, W), jnp.float32)),
        mesh=_sc_mesh())
    def gather_kernel(xl_hbm, xr_hbm, i_hbm, ol_hbm, or_hbm):
        cid = jax.lax.axis_index("c")
        nblk = B // SCW

        def gath(x_hbm, o_hbm):
            def body(i_vmem, o_vmem):
                pltpu.sync_copy(x_hbm.at[i_vmem.at[0]], o_vmem)

            pltpu.emit_pipeline(
                body,
                grid=(P // SCW,),
                in_specs=[pl.BlockSpec(
                    (1, SCW),
                    lambda i: (jax.lax.div(i, nblk), jax.lax.rem(i, nblk)))],
                out_specs=[pl.BlockSpec((SCW, W), lambda i: (i, 0))],
                core_axis_name="s",
                dimension_semantics=(pltpu.PARALLEL,),
            )(i_hbm, o_hbm)

        @pl.when(cid == 0)
        def _():
            gath(xl_hbm, ol_hbm)

        @pl.when(cid == 1)
        def _():
            gath(xr_hbm, or_hbm)

    return gather_kernel(bufl, bufr, pos2)


def _expert_body(te_ref, bufl_ref, bufr_ref, w1_ref, b1_ref, w2_ref, b2_ref,
                 ol_ref, or_ref):
    xt = jnp.concatenate([bufl_ref[...], bufr_ref[...]], axis=1)
    xt = xt.astype(w1_ref.dtype)
    h = jnp.dot(xt, w1_ref[0], preferred_element_type=jnp.float32)
    h = jnp.maximum(h + b1_ref[0], 0.0).astype(w1_ref.dtype)
    o = jnp.dot(h, w2_ref[0], preferred_element_type=jnp.float32)
    o = o + b2_ref[0]
    m = jnp.max(o, axis=1, keepdims=True)
    p = jnp.exp(o - m)
    o = p / jnp.sum(p, axis=1, keepdims=True)
    half = o.shape[1] // 2
    ol_ref[...] = o[:, :half]
    or_ref[...] = o[:, half:]


def _expert_mlp(te, bufl, bufr, W1, b1, W2, b2, nt):
    npad, Dh = bufl.shape
    E, D, H = W1.shape
    MOUT = W2.shape[2]
    grid_spec = pltpu.PrefetchScalarGridSpec(
        num_scalar_prefetch=1,
        grid=(nt,),
        in_specs=[
            pl.BlockSpec((TILE, Dh), lambda i, te_r: (i, 0)),
            pl.BlockSpec((TILE, Dh), lambda i, te_r: (i, 0)),
            pl.BlockSpec((1, D, H), lambda i, te_r: (te_r[i], 0, 0)),
            pl.BlockSpec((1, 1, H), lambda i, te_r: (te_r[i], 0, 0)),
            pl.BlockSpec((1, H, MOUT), lambda i, te_r: (te_r[i], 0, 0)),
            pl.BlockSpec((1, 1, MOUT), lambda i, te_r: (te_r[i], 0, 0)),
        ],
        out_specs=[
            pl.BlockSpec((TILE, MOUT // 2), lambda i, te_r: (i, 0)),
            pl.BlockSpec((TILE, MOUT // 2), lambda i, te_r: (i, 0)),
        ],
    )
    return pl.pallas_call(
        _expert_body,
        grid_spec=grid_spec,
        out_shape=(
            jax.ShapeDtypeStruct((npad, MOUT // 2), jnp.float32),
            jax.ShapeDtypeStruct((npad, MOUT // 2), jnp.float32),
        ),
    )(te, bufl, bufr, W1, b1, W2, b2)


def _combine_heads_body(opl_ref, opr_ref, g_ref, w11_ref, b11_ref, w12_ref,
                        b12_ref, w21_ref, b21_ref, w22_ref, b22_ref,
                        y1_ref, y2_ref):
    g = g_ref[...]
    op0 = jnp.concatenate([opl_ref[0], opr_ref[0]], axis=1)
    op1 = jnp.concatenate([opl_ref[1], opr_ref[1]], axis=1)
    moe = op0 * g[:, 0:1] + op1 * g[:, 1:2]
    mo = moe.astype(w11_ref.dtype)
    h1 = jnp.dot(mo, w11_ref[...], preferred_element_type=jnp.float32)
    h1 = jnp.maximum(h1 + b11_ref[...], 0.0).astype(w11_ref.dtype)
    y1_ref[...] = (jnp.dot(h1, w12_ref[...],
                           preferred_element_type=jnp.float32) + b12_ref[...])
    h2 = jnp.dot(mo, w21_ref[...], preferred_element_type=jnp.float32)
    h2 = jnp.maximum(h2 + b21_ref[...], 0.0).astype(w11_ref.dtype)
    y2_ref[...] = (jnp.dot(h2, w22_ref[...],
                           preferred_element_type=jnp.float32) + b22_ref[...])


def _combine_heads(opl, opr, gT, hw11, hb11, hw12, hb12, hw21, hb21,
                   hw22, hb22):
    _, B, MOUTH = opl.shape
    MOUT = 2 * MOUTH
    MH = hw11.shape[1]
    OUT = hw12.shape[1]
    TT = 512
    grid = (B // TT,)
    y1, y2 = pl.pallas_call(
        _combine_heads_body,
        grid=grid,
        in_specs=[
            pl.BlockSpec((2, TT, MOUTH), lambda i: (0, i, 0)),
            pl.BlockSpec((2, TT, MOUTH), lambda i: (0, i, 0)),
            pl.BlockSpec((TT, 2), lambda i: (i, 0)),
            pl.BlockSpec((MOUT, MH), lambda i: (0, 0)),
            pl.BlockSpec((1, MH), lambda i: (0, 0)),
            pl.BlockSpec((MH, OUT), lambda i: (0, 0)),
            pl.BlockSpec((1, OUT), lambda i: (0, 0)),
            pl.BlockSpec((MOUT, MH), lambda i: (0, 0)),
            pl.BlockSpec((1, MH), lambda i: (0, 0)),
            pl.BlockSpec((MH, OUT), lambda i: (0, 0)),
            pl.BlockSpec((1, OUT), lambda i: (0, 0)),
        ],
        out_specs=[
            pl.BlockSpec((TT, OUT), lambda i: (i, 0)),
            pl.BlockSpec((TT, OUT), lambda i: (i, 0)),
        ],
        out_shape=(
            jax.ShapeDtypeStruct((B, OUT), jnp.float32),
            jax.ShapeDtypeStruct((B, OUT), jnp.float32),
        ),
    )(opl, opr, gT, hw11, hb11, hw12, hb12, hw21, hb21, hw22, hb22)
    return y1, y2


def kernel(x, w_gate, W1, b1, W2, b2, m1_W1, m1_b1, m1_W2, m1_b2,
           m2_W1, m2_b1, m2_W2, m2_b2):
    B, D = x.shape
    E = w_gate.shape[1]
    H = W1.shape[2]
    MOUT = W2.shape[2]
    MH = m1_W1.shape[1]
    OUT = m1_W2.shape[1]
    npad = ((2 * B + E * (TILE - 1)) + TILE - 1) // TILE * TILE
    nt = npad // TILE
    nt_pad = ((nt + 63) // 64) * 64

    gates, pos, te = _gate_route(x, w_gate.T, nt_pad)
    te1 = te.reshape(nt_pad)[:nt]

    bufl, bufr = _sc_scatter_rows(x, pos, npad)
    obufl, obufr = _expert_mlp(
        te1, bufl, bufr, W1, b1.reshape(E, 1, H), W2, b2.reshape(E, 1, MOUT),
        nt)
    opl, opr = _sc_gather_rows(obufl, obufr, pos)

    y1, y2 = _combine_heads(
        opl.reshape(2, B, MOUT // 2), opr.reshape(2, B, MOUT // 2), gates,
        m1_W1, m1_b1.reshape(1, MH), m1_W2, m1_b2.reshape(1, OUT),
        m2_W1, m2_b1.reshape(1, MH), m2_W2, m2_b2.reshape(1, OUT))
    return (y1, y2)


# speedup vs baseline: 3.7722x; 1.1071x over previous
import functools

import jax
import jax.numpy as jnp
from jax.experimental import pallas as pl
from jax.experimental.pallas import tpu as pltpu
from jax.experimental.pallas import tpu_sc as plsc

TILE = 256
SCW = 128


def _gate_route_body(x_ref, wgt_ref, gates_ref, pos_ref, te_ref):
    B, D = x_ref.shape
    E = wgt_ref.shape[0]
    P = 2 * B
    logitsT = jax.lax.dot_general(
        wgt_ref[...], x_ref[...], (((1,), (1,)), ((), ())),
        preferred_element_type=jnp.float32)
    NEG = jnp.float32(-1e30)
    m1 = jnp.full((1, B), NEG, jnp.float32)
    a1 = jnp.zeros((1, B), jnp.int32)
    for e in range(E):
        v = logitsT[e:e + 1, :]
        take = v > m1
        a1 = jnp.where(take, e, a1)
        m1 = jnp.where(take, v, m1)
    m2 = jnp.full((1, B), NEG, jnp.float32)
    a2 = jnp.zeros((1, B), jnp.int32)
    for e in range(E):
        v = logitsT[e:e + 1, :]
        take = (v > m2) & (a1 != e)
        a2 = jnp.where(take, e, a2)
        m2 = jnp.where(take, v, m2)
    g1 = 1.0 / (1.0 + jnp.exp(m2 - m1))
    gates_ref[...] = jnp.transpose(
        jnp.concatenate([g1, 1.0 - g1], axis=0))

    rows = []
    for e in range(E):
        rows.append(jnp.concatenate(
            [(a1 == e), (a2 == e)], axis=1).astype(jnp.float32))
    oh = jnp.concatenate(rows, axis=0)

    r = oh
    k = 1
    while k < P:
        r = r + jnp.concatenate(
            [jnp.zeros((E, k), jnp.float32), r[:, :P - k]], axis=1)
        k *= 2
    c = r[:, P - 1:P]
    pc = jnp.floor((c + (TILE - 1)) / TILE) * TILE
    inc = pc
    for k in (1, 2, 4):
        inc = inc + jnp.concatenate(
            [jnp.zeros((k, 1), jnp.float32), inc[:E - k]], axis=0)
    poff = inc - pc
    posf = jnp.sum(oh * (r - 1.0 + poff), axis=0, keepdims=True)
    pos_ref[...] = jnp.concatenate(
        [posf[:, :B], posf[:, B:]], axis=0).astype(jnp.int32)

    pend = poff + pc
    lanes = jax.lax.broadcasted_iota(jnp.int32, (1, te_ref.shape[1]), 1)
    starts = lanes.astype(jnp.float32) * TILE
    tecnt = jnp.sum((pend <= starts).astype(jnp.int32), axis=0, keepdims=True)
    nact = (jnp.sum(pc, axis=0, keepdims=True) / TILE).astype(jnp.int32)
    te_ref[...] = jnp.where(lanes == te_ref.shape[1] - 1, nact,
                            jnp.minimum(tecnt, E - 1))


def _gate_route(x, w_gateT, nt_pad):
    B = x.shape[0]
    E = w_gateT.shape[0]
    return pl.pallas_call(
        _gate_route_body,
        out_shape=(
            jax.ShapeDtypeStruct((B, 2), jnp.float32),
            jax.ShapeDtypeStruct((2, B), jnp.int32),
            jax.ShapeDtypeStruct((1, nt_pad), jnp.int32),
        ),
    )(x, w_gateT)


def _sc_mesh():
    return plsc.VectorSubcoreMesh(core_axis_name="c", subcore_axis_name="s")


def _sc_scatter_rows(x, pos2, npad):
    B, D = x.shape
    W = D // 2

    @functools.partial(
        pl.kernel,
        out_type=(jax.ShapeDtypeStruct((npad, W), jnp.float32),
                  jax.ShapeDtypeStruct((npad, W), jnp.float32)),
        mesh=_sc_mesh())
    def scatter_kernel(x_hbm, i_hbm, ol_hbm, or_hbm):
        cid = jax.lax.axis_index("c")

        def scat(col, o_hbm):
            def body(x_vmem, i_vmem):
                pltpu.sync_copy(x_vmem, o_hbm.at[i_vmem.at[0]])
                pltpu.sync_copy(x_vmem, o_hbm.at[i_vmem.at[1]])

            pltpu.emit_pipeline(
                body,
                grid=(B // SCW,),
                in_specs=[
                    pl.BlockSpec((SCW, W), lambda i: (i, col)),
                    pl.BlockSpec((2, SCW), lambda i: (0, i)),
                ],
                out_specs=[],
                core_axis_name="s",
                dimension_semantics=(pltpu.PARALLEL,),
            )(x_hbm, i_hbm)

        @pl.when(cid == 0)
        def _():
            scat(0, ol_hbm)

        @pl.when(cid == 1)
        def _():
            scat(1, or_hbm)

    return scatter_kernel(x, pos2)


def _sc_gather_rows(bufl, bufr, pos2):
    W = bufl.shape[1]
    K, B = pos2.shape
    P = K * B

    @functools.partial(
        pl.kernel,
        out_type=(jax.ShapeDtypeStruct((P, W), jnp.float32),
                  jax.ShapeDtypeStruct((P, W), jnp.float32)),
        mesh=_sc_mesh())
    def gather_kernel(xl_hbm, xr_hbm, i_hbm, ol_hbm, or_hbm):
        cid = jax.lax.axis_index("c")
        nblk = B // SCW

        def gath(x_hbm, o_hbm):
            def body(i_vmem, o_vmem):
                pltpu.sync_copy(x_hbm.at[i_vmem.at[0]], o_vmem)

            pltpu.emit_pipeline(
                body,
                grid=(P // SCW,),
                in_specs=[pl.BlockSpec(
                    (1, SCW),
                    lambda i: (jax.lax.div(i, nblk), jax.lax.rem(i, nblk)))],
                out_specs=[pl.BlockSpec((SCW, W), lambda i: (i, 0))],
                core_axis_name="s",
                dimension_semantics=(pltpu.PARALLEL,),
            )(i_hbm, o_hbm)

        @pl.when(cid == 0)
        def _():
            gath(xl_hbm, ol_hbm)

        @pl.when(cid == 1)
        def _():
            gath(xr_hbm, or_hbm)

    return gather_kernel(bufl, bufr, pos2)


def _expert_body(te_ref, nact_ref, bufl_ref, bufr_ref, w1_ref, b1_ref,
                 w2_ref, b2_ref, ol_ref, or_ref):
    @pl.when(pl.program_id(0) < nact_ref[0])
    def _():
        xt = jnp.concatenate([bufl_ref[...], bufr_ref[...]], axis=1)
        xt = xt.astype(w1_ref.dtype)
        h = jnp.dot(xt, w1_ref[0], preferred_element_type=jnp.float32)
        h = jnp.maximum(h + b1_ref[0], 0.0).astype(w1_ref.dtype)
        o = jnp.dot(h, w2_ref[0], preferred_element_type=jnp.float32)
        o = o + b2_ref[0]
        m = jnp.max(o, axis=1, keepdims=True)
        p = jnp.exp(o - m)
        o = p / jnp.sum(p, axis=1, keepdims=True)
        half = o.shape[1] // 2
        ol_ref[...] = o[:, :half]
        or_ref[...] = o[:, half:]


def _expert_mlp(te, nact, bufl, bufr, W1, b1, W2, b2, nt):
    npad, Dh = bufl.shape
    E, D, H = W1.shape
    MOUT = W2.shape[2]
    grid_spec = pltpu.PrefetchScalarGridSpec(
        num_scalar_prefetch=2,
        grid=(nt,),
        in_specs=[
            pl.BlockSpec((TILE, Dh), lambda i, te_r, na: (i, 0)),
            pl.BlockSpec((TILE, Dh), lambda i, te_r, na: (i, 0)),
            pl.BlockSpec((1, D, H), lambda i, te_r, na: (te_r[i], 0, 0)),
            pl.BlockSpec((1, 1, H), lambda i, te_r, na: (te_r[i], 0, 0)),
            pl.BlockSpec((1, H, MOUT), lambda i, te_r, na: (te_r[i], 0, 0)),
            pl.BlockSpec((1, 1, MOUT), lambda i, te_r, na: (te_r[i], 0, 0)),
        ],
        out_specs=[
            pl.BlockSpec((TILE, MOUT // 2), lambda i, te_r, na: (i, 0)),
            pl.BlockSpec((TILE, MOUT // 2), lambda i, te_r, na: (i, 0)),
        ],
    )
    return pl.pallas_call(
        _expert_body,
        grid_spec=grid_spec,
        out_shape=(
            jax.ShapeDtypeStruct((npad, MOUT // 2), jnp.float32),
            jax.ShapeDtypeStruct((npad, MOUT // 2), jnp.float32),
        ),
    )(te, nact, bufl, bufr, W1, b1, W2, b2)


def _combine_heads_body(opl_ref, opr_ref, g_ref, w11_ref, b11_ref, w12_ref,
                        b12_ref, w21_ref, b21_ref, w22_ref, b22_ref,
                        y1_ref, y2_ref):
    g = g_ref[...]
    op0 = jnp.concatenate([opl_ref[0], opr_ref[0]], axis=1)
    op1 = jnp.concatenate([opl_ref[1], opr_ref[1]], axis=1)
    moe = op0 * g[:, 0:1] + op1 * g[:, 1:2]
    mo = moe.astype(w11_ref.dtype)
    h1 = jnp.dot(mo, w11_ref[...], preferred_element_type=jnp.float32)
    h1 = jnp.maximum(h1 + b11_ref[...], 0.0).astype(w11_ref.dtype)
    y1_ref[...] = (jnp.dot(h1, w12_ref[...],
                           preferred_element_type=jnp.float32) + b12_ref[...])
    h2 = jnp.dot(mo, w21_ref[...], preferred_element_type=jnp.float32)
    h2 = jnp.maximum(h2 + b21_ref[...], 0.0).astype(w11_ref.dtype)
    y2_ref[...] = (jnp.dot(h2, w22_ref[...],
                           preferred_element_type=jnp.float32) + b22_ref[...])


def _combine_heads(opl, opr, gT, hw11, hb11, hw12, hb12, hw21, hb21,
                   hw22, hb22):
    _, B, MOUTH = opl.shape
    MOUT = 2 * MOUTH
    MH = hw11.shape[1]
    OUT = hw12.shape[1]
    TT = 512
    grid = (B // TT,)
    y1, y2 = pl.pallas_call(
        _combine_heads_body,
        grid=grid,
        in_specs=[
            pl.BlockSpec((2, TT, MOUTH), lambda i: (0, i, 0)),
            pl.BlockSpec((2, TT, MOUTH), lambda i: (0, i, 0)),
            pl.BlockSpec((TT, 2), lambda i: (i, 0)),
            pl.BlockSpec((MOUT, MH), lambda i: (0, 0)),
            pl.BlockSpec((1, MH), lambda i: (0, 0)),
            pl.BlockSpec((MH, OUT), lambda i: (0, 0)),
            pl.BlockSpec((1, OUT), lambda i: (0, 0)),
            pl.BlockSpec((MOUT, MH), lambda i: (0, 0)),
            pl.BlockSpec((1, MH), lambda i: (0, 0)),
            pl.BlockSpec((MH, OUT), lambda i: (0, 0)),
            pl.BlockSpec((1, OUT), lambda i: (0, 0)),
        ],
        out_specs=[
            pl.BlockSpec((TT, OUT), lambda i: (i, 0)),
            pl.BlockSpec((TT, OUT), lambda i: (i, 0)),
        ],
        out_shape=(
            jax.ShapeDtypeStruct((B, OUT), jnp.float32),
            jax.ShapeDtypeStruct((B, OUT), jnp.float32),
        ),
    )(opl, opr, gT, hw11, hb11, hw12, hb12, hw21, hb21, hw22, hb22)
    return y1, y2


def kernel(x, w_gate, W1, b1, W2, b2, m1_W1, m1_b1, m1_W2, m1_b2,
           m2_W1, m2_b1, m2_W2, m2_b2):
    B, D = x.shape
    E = w_gate.shape[1]
    H = W1.shape[2]
    MOUT = W2.shape[2]
    MH = m1_W1.shape[1]
    OUT = m1_W2.shape[1]
    npad = ((2 * B + E * (TILE - 1)) + TILE - 1) // TILE * TILE
    nt = npad // TILE
    nt_pad = ((nt + 63) // 64) * 64

    gates, pos, te = _gate_route(x, w_gate.T, nt_pad)
    tef = te.reshape(nt_pad)
    te1 = tef[:nt]
    nact = tef[nt_pad - 1:nt_pad]

    bufl, bufr = _sc_scatter_rows(x, pos, npad)
    obufl, obufr = _expert_mlp(
        te1, nact, bufl, bufr, W1, b1.reshape(E, 1, H),
        W2, b2.reshape(E, 1, MOUT), nt)
    opl, opr = _sc_gather_rows(obufl, obufr, pos)

    y1, y2 = _combine_heads(
        opl.reshape(2, B, MOUT // 2), opr.reshape(2, B, MOUT // 2), gates,
        m1_W1, m1_b1.reshape(1, MH), m1_W2, m1_b2.reshape(1, OUT),
        m2_W1, m2_b1.reshape(1, MH), m2_W2, m2_b2.reshape(1, OUT))
    return (y1, y2)
